# baseline (speedup 1.0000x reference)
import jax
import jax.numpy as jnp
from jax import lax
from jax.experimental import pallas as pl
from jax.experimental.pallas import tpu as pltpu
from jax.experimental.pallas import tpu_sc as plsc

B = 16384
D = 128
S = 11
NUM_SAMPLE = 10
NW = 32
GROUP = 128
G_PER_W = B // (NW * GROUP)
B_PER_W = G_PER_W * GROUP
HALF = 2 * GROUP
ACC_ROWS = 16 * HALF
LANES = 16
INV = 1.0 / S


def _agg_body(ids_hbm, feat_hbm, out_hbm, idx_v, scat, rows, gsems, ssems,
              shared):
  cid = lax.axis_index("c")
  sid = lax.axis_index("s")
  wid = sid * 2 + cid
  gbase = wid * G_PER_W
  lbase = sid * HALF

  for j in range(S):
    pltpu.sync_copy(ids_hbm.at[j, pl.ds(gbase, G_PER_W)], idx_v.at[j])

  iota = lax.iota(jnp.int32, LANES)
  for k in range(2):
    for c in range(D // LANES):
      scat[k, pl.ds(c * LANES, LANES)] = lbase + k * GROUP + c * LANES + iota

  def issue(j, g, b):
    pltpu.async_copy(feat_hbm.at[idx_v.at[j, g]], rows[b], gsems[b])

  def drain_g(b):
    pltpu.make_async_copy(feat_hbm.at[idx_v.at[0, 0]], rows[b],
                          gsems[b]).wait()

  def start_scat(k, b):
    pltpu.async_copy(rows[b], shared.at[scat.at[k]], ssems[b], add=True)

  def drain_s(k, b):
    pltpu.make_async_copy(rows[b], shared.at[scat.at[k]], ssems[b]).wait()

  def half_body(h, _):
    gs = (h * 2, h * 2 + 1)

    for k in range(2):
      issue(jnp.int32(0), gs[k], 2 * k)
      issue(jnp.int32(1), gs[k], 2 * k + 1)

    for k in range(2):
      drain_g(2 * k)
      pltpu.sync_copy(rows[2 * k], shared.at[pl.ds(lbase + k * GROUP, GROUP)])
      issue(jnp.int32(2), gs[k], 2 * k)

    for k in range(2):
      drain_g(2 * k + 1)
      start_scat(k, 2 * k + 1)

    def tbody(t, _):
      jo = 2 * t + 3
      jn = 2 * t + 4

      for k in range(2):
        drain_s(k, 2 * k + 1)

        @pl.when(jo < S)
        def _(k=k):
          issue(jo, gs[k], 2 * k + 1)

        drain_g(2 * k)
        start_scat(k, 2 * k)

      for k in range(2):
        @pl.when(jo < S)
        def _(k=k):
          drain_s(k, 2 * k)

          @pl.when(jn < S)
          def _():
            issue(jn, gs[k], 2 * k)

          drain_g(2 * k + 1)
          start_scat(k, 2 * k + 1)

      return 0

    lax.fori_loop(0, 5, tbody, 0)

    obase = wid * B_PER_W + h * HALF
    for k in range(2):
      drain_s(k, 2 * k)
      pltpu.sync_copy(shared.at[pl.ds(lbase + k * GROUP, GROUP)], rows[2 * k])

      def sbody(r, _, k=k):
        for c in range(D // LANES):
          sl = pl.ds(c * LANES, LANES)
          rows[2 * k][r, sl] = rows[2 * k][r, sl] * INV
        return 0

      lax.fori_loop(0, GROUP, sbody, 0)
      pltpu.async_copy(
          rows[2 * k], out_hbm.at[pl.ds(obase + k * GROUP, GROUP)],
          gsems[2 * k])

    for k in range(2):
      pltpu.make_async_copy(
          rows[2 * k], out_hbm.at[pl.ds(obase + k * GROUP, GROUP)],
          gsems[2 * k]).wait()
    return 0

  lax.fori_loop(0, 2, half_body, 0)


@jax.jit
def _agg(ids_r, features):
  mesh = plsc.VectorSubcoreMesh(core_axis_name="c", subcore_axis_name="s")
  return pl.kernel(
      _agg_body,
      out_type=jax.ShapeDtypeStruct((B, D), jnp.float32),
      mesh=mesh,
      scratch_types=[
          pltpu.VMEM((S, G_PER_W, GROUP), jnp.int32),
          pltpu.VMEM((2, GROUP), jnp.int32),
          [pltpu.VMEM((GROUP, D), jnp.float32)] * 4,
          [pltpu.SemaphoreType.DMA] * 4,
          [pltpu.SemaphoreType.DMA] * 4,
          pltpu.VMEM_SHARED((ACC_ROWS, D), jnp.float32),
      ],
  )(ids_r, features)


def kernel(nodes, neighbours_full, features):
  all_ids = jnp.concatenate(
      [nodes[:, None], neighbours_full[:, :NUM_SAMPLE]], axis=1)
  ids_r = all_ids.T.reshape(S, B // GROUP, GROUP)
  return _agg(ids_r, features)


# speedup vs baseline: 1.0640x; 1.0640x over previous
import jax
import jax.numpy as jnp
from jax import lax
from jax.experimental import pallas as pl
from jax.experimental.pallas import tpu as pltpu
from jax.experimental.pallas import tpu_sc as plsc

B = 16384
D = 128
S = 11
NUM_SAMPLE = 10
NW = 32
GROUP = 64
G_PER_W = B // (NW * GROUP)
B_PER_W = G_PER_W * GROUP
NSG = 4
HALF = NSG * GROUP
ACC_ROWS = 16 * HALF
LANES = 16
INV = 1.0 / S


def _agg_body(ids_hbm, feat_hbm, out_hbm, idx_v, scat, rows, sems, shared):
  cid = lax.axis_index("c")
  sid = lax.axis_index("s")
  wid = sid * 2 + cid
  gbase = wid * G_PER_W
  lbase = sid * HALF

  for j in range(S):
    pltpu.sync_copy(ids_hbm.at[j, pl.ds(gbase, G_PER_W)], idx_v.at[j])

  iota = lax.iota(jnp.int32, LANES)
  for sg in range(NSG):
    for c in range(GROUP // LANES):
      scat[sg, pl.ds(c * LANES, LANES)] = lbase + sg * GROUP + c * LANES + iota

  def issue(j, g, b):
    pltpu.async_copy(feat_hbm.at[idx_v.at[j, g]], rows[b], sems[b])

  def drain(b):
    pltpu.make_async_copy(feat_hbm.at[idx_v.at[0, 0]], rows[b], sems[b]).wait()

  def half_body(h, _):
    gg = [h * NSG + sg for sg in range(NSG)]

    for sg in range(NSG):
      issue(jnp.int32(0), gg[sg], sg)
    for sg in range(NSG):
      issue(jnp.int32(1), gg[sg], NSG + sg)

    for sg in range(NSG):
      drain(sg)
      pltpu.sync_copy(rows[sg], shared.at[pl.ds(lbase + sg * GROUP, GROUP)])
      issue(jnp.int32(2), gg[sg], sg)

    def tbody(t, _):
      ja = 2 * t + 3
      jb = 2 * t + 4

      for sg in range(NSG):
        drain(NSG + sg)
        pltpu.sync_copy(rows[NSG + sg], shared.at[scat.at[sg]], add=True)

        @pl.when(ja < S)
        def _(sg=sg):
          issue(ja, gg[sg], NSG + sg)

      for sg in range(NSG):
        drain(sg)
        pltpu.sync_copy(rows[sg], shared.at[scat.at[sg]], add=True)

        @pl.when(jb < S)
        def _(sg=sg):
          issue(jb, gg[sg], sg)

      return 0

    lax.fori_loop(0, 5, tbody, 0)

    obase = wid * B_PER_W + h * HALF
    for sg in range(NSG):
      pltpu.sync_copy(shared.at[pl.ds(lbase + sg * GROUP, GROUP)], rows[sg])

      def sbody(r, _, sg=sg):
        for c in range(D // LANES):
          sl = pl.ds(c * LANES, LANES)
          rows[sg][r, sl] = rows[sg][r, sl] * INV
        return 0

      lax.fori_loop(0, GROUP, sbody, 0)
      pltpu.async_copy(
          rows[sg], out_hbm.at[pl.ds(obase + sg * GROUP, GROUP)], sems[sg])

    for sg in range(NSG):
      pltpu.make_async_copy(
          rows[sg], out_hbm.at[pl.ds(obase + sg * GROUP, GROUP)],
          sems[sg]).wait()
    return 0

  lax.fori_loop(0, 2, half_body, 0)


@jax.jit
def _agg(ids_r, features):
  mesh = plsc.VectorSubcoreMesh(core_axis_name="c", subcore_axis_name="s")
  return pl.kernel(
      _agg_body,
      out_type=jax.ShapeDtypeStruct((B, D), jnp.float32),
      mesh=mesh,
      scratch_types=[
          pltpu.VMEM((S, G_PER_W, GROUP), jnp.int32),
          pltpu.VMEM((NSG, GROUP), jnp.int32),
          [pltpu.VMEM((GROUP, D), jnp.float32)] * 8,
          [pltpu.SemaphoreType.DMA] * 8,
          pltpu.VMEM_SHARED((ACC_ROWS, D), jnp.float32),
      ],
  )(ids_r, features)


def kernel(nodes, neighbours_full, features):
  all_ids = jnp.concatenate(
      [nodes[:, None], neighbours_full[:, :NUM_SAMPLE]], axis=1)
  ids_r = all_ids.T.reshape(S, B // GROUP, GROUP)
  return _agg(ids_r, features)
